# Initial kernel scaffold; baseline (speedup 1.0000x reference)
import jax
import jax.numpy as jnp
from jax.experimental import pallas as pl


def kernel(concatenated_node_features, interaction_feature, adj_mat, line_adj_mat, nenn_edge_index, object_pairs, num_obj, nenn_num_edges, W_node, W_edge, lr_W1, lr_b1, lr_W2, lr_b2, scr_W1, scr_b1, scr_W2, scr_b2, mr_W1, mr_b1, mr_W2, mr_b2):
    raise NotImplementedError("write your pallas kernel here")


# speedup vs baseline: 14.8065x; 14.8065x over previous
import jax
import jax.numpy as jnp
import numpy as np
from jax.experimental import pallas as pl
from jax.experimental.pallas import tpu as pltpu

B, N, E, DN, DE = 8, 128, 1024, 256, 16
WN, WE = 64, 64
EMB = WN + WE
P = 64
CIN = 2 * EMB
ODIMS = (10, 6, 5)
OSUM = sum(ODIMS)
KMAX = 64


def _nenn_kernel(nf_ref, adj_ref, ladj_ref, ei_ref, eit_ref, ef_ref, pairs_ref,
                 nmask_row_ref, nmask_col_ref, emask_row_ref, emask_col_ref,
                 Wn_ref, We_ref, W1_ref, b1_ref, W2_ref, b2_ref, out_ref):
    f32 = jnp.float32
    nf = nf_ref[0]
    adj = adj_ref[0]
    nmask_row = nmask_row_ref[0]
    nmask_col = nmask_col_ref[0]
    emask_row = emask_row_ref[0]
    emask_col = emask_col_ref[0]

    hn = jnp.dot(nf, Wn_ref[...], preferred_element_type=f32)
    adj_m = adj * nmask_row
    nn_agg = jnp.dot(adj_m, hn, preferred_element_type=f32) / (
        jnp.sum(adj_m, axis=1, keepdims=True) + 1e-6)

    src_col = eit_ref[0, :, 0:1]
    dst_col = eit_ref[0, :, 1:2]
    key = src_col * KMAX + dst_col
    g = jnp.zeros((E, DE), f32)
    for c in range(4):
        ids = jax.lax.broadcasted_iota(jnp.int32, (1, 1024), 1) + c * 1024
        oh = (key == ids).astype(f32)
        g = g + jnp.dot(oh, ef_ref[0, c * 1024:(c + 1) * 1024, :],
                        preferred_element_type=f32)
    he = jnp.dot(g, We_ref[...], preferred_element_type=f32) * emask_col

    src_row = ei_ref[0, 0:1, :]
    dst_row = ei_ref[0, 1:2, :]
    iota_n = jax.lax.broadcasted_iota(jnp.int32, (N, 1), 0)
    SD = ((iota_n == src_row).astype(f32) +
          (iota_n == dst_row).astype(f32)) * emask_row
    deg = jnp.sum(SD, axis=1, keepdims=True)
    inc = jnp.dot(SD, he, preferred_element_type=f32) / (deg + 1e-6)

    node_emb = jnp.concatenate(
        [jax.nn.relu(nn_agg), jax.nn.relu(inc)], axis=1) * nmask_col

    ladj_m = ladj_ref[0] * emask_row
    line_agg = jnp.dot(ladj_m, he, preferred_element_type=f32) / (
        jnp.sum(ladj_m, axis=1, keepdims=True) + 1e-6)

    iota_k = jax.lax.broadcasted_iota(jnp.int32, (1, KMAX), 1)
    OH_ep = ((src_col == iota_k).astype(f32) +
             (dst_col == iota_k).astype(f32))
    ep = 0.5 * jnp.dot(OH_ep, hn[:KMAX, :], preferred_element_type=f32)

    ee = jnp.concatenate([jax.nn.relu(ep), jax.nn.relu(line_agg)], axis=1)

    i0 = pairs_ref[0, :, 0:1]
    i1 = pairs_ref[0, :, 1:2]
    M = ((i0 == src_row).astype(f32) * (i1 == dst_row).astype(f32)) * emask_row
    ee_pair = jnp.dot(M, ee, preferred_element_type=f32)

    iota_nr = jax.lax.broadcasted_iota(jnp.int32, (1, N), 1)
    O = (i0 == iota_nr).astype(f32) + (i1 == iota_nr).astype(f32)
    pair_emb = jnp.dot(O, node_emb, preferred_element_type=f32)

    cls_in = jnp.concatenate([pair_emb, ee_pair], axis=1)
    h = jax.nn.relu(jnp.dot(cls_in, W1_ref[...], preferred_element_type=f32)
                    + b1_ref[...])
    out = jnp.dot(h, W2_ref[...], preferred_element_type=f32) + b2_ref[...]
    out_ref[0] = out


def kernel(concatenated_node_features, interaction_feature, adj_mat,
           line_adj_mat, nenn_edge_index, object_pairs, num_obj,
           nenn_num_edges, W_node, W_edge, lr_W1, lr_b1, lr_W2, lr_b2,
           scr_W1, scr_b1, scr_W2, scr_b2, mr_W1, mr_b1, mr_W2, mr_b2):
    f32 = jnp.float32
    ef_flat = interaction_feature[:, :KMAX, :KMAX, :].reshape(B, KMAX * KMAX, DE)
    ei_t = jnp.transpose(nenn_edge_index, (0, 2, 1))

    nvec = jnp.arange(N, dtype=jnp.int32)
    evec = jnp.arange(E, dtype=jnp.int32)
    nmask_row = (nvec[None, None, :] < num_obj[:, None, None]).astype(f32)
    nmask_col = (nvec[None, :, None] < num_obj[:, None, None]).astype(f32)
    emask_row = (evec[None, None, :] < nenn_num_edges[:, None, None]).astype(f32)
    emask_col = (evec[None, :, None] < nenn_num_edges[:, None, None]).astype(f32)

    W1cat = jnp.concatenate([lr_W1, scr_W1, mr_W1], axis=1)
    b1cat = jnp.concatenate([lr_b1, scr_b1, mr_b1])[None, :]
    z = jnp.zeros
    W2blk = jnp.concatenate([
        jnp.concatenate([lr_W2, z((128, ODIMS[1]), f32), z((128, ODIMS[2]), f32)], 1),
        jnp.concatenate([z((128, ODIMS[0]), f32), scr_W2, z((128, ODIMS[2]), f32)], 1),
        jnp.concatenate([z((128, ODIMS[0]), f32), z((128, ODIMS[1]), f32), mr_W2], 1),
    ], axis=0)
    b2cat = jnp.concatenate([lr_b2, scr_b2, mr_b2])[None, :]

    def bspec(shape):
        return pl.BlockSpec((1,) + shape, lambda b: (b, 0, 0)[:1 + len(shape)])

    def wspec(shape):
        nd = len(shape)
        return pl.BlockSpec(shape, lambda b: (0,) * nd)

    out = pl.pallas_call(
        _nenn_kernel,
        grid=(B,),
        in_specs=[
            bspec((N, DN)), bspec((N, N)), bspec((E, E)), bspec((2, E)),
            bspec((E, 2)), bspec((KMAX * KMAX, DE)), bspec((P, 2)),
            bspec((1, N)), bspec((N, 1)), bspec((1, E)), bspec((E, 1)),
            wspec((DN, WN)), wspec((DE, WE)), wspec((CIN, 384)),
            wspec((1, 384)), wspec((384, OSUM)), wspec((1, OSUM)),
        ],
        out_specs=bspec((P, OSUM)),
        out_shape=jax.ShapeDtypeStruct((B, P, OSUM), f32),
        compiler_params=pltpu.CompilerParams(
            dimension_semantics=("arbitrary",)),
    )(concatenated_node_features, adj_mat, line_adj_mat, nenn_edge_index,
      ei_t, ef_flat, object_pairs, nmask_row, nmask_col, emask_row,
      emask_col, W_node, W_edge, W1cat, b1cat, W2blk, b2cat)

    lr = out[:, :, 0:ODIMS[0]]
    cr = out[:, :, ODIMS[0]:ODIMS[0] + ODIMS[1]]
    mr = out[:, :, ODIMS[0] + ODIMS[1]:OSUM]
    return (lr, cr, mr)


# speedup vs baseline: 15.2509x; 1.0300x over previous
import jax
import jax.numpy as jnp
import numpy as np
from jax.experimental import pallas as pl
from jax.experimental.pallas import tpu as pltpu

B, N, E, DN, DE = 8, 128, 1024, 256, 16
WN, WE = 64, 64
EMB = WN + WE
P = 64
CIN = 2 * EMB
ODIMS = (10, 6, 5)
OSUM = sum(ODIMS)
KMAX = 64


def _nenn_kernel(no_ref, ne_ref, nf_ref, adj_ref, ladj_ref, ei_ref, eit_ref,
                 ef_ref, pairs_ref, Wn_ref, We_ref,
                 W1a_ref, b1a_ref, W2a_ref, b2a_ref,
                 W1b_ref, b1b_ref, W2b_ref, b2b_ref,
                 W1c_ref, b1c_ref, W2c_ref, b2c_ref,
                 outa_ref, outb_ref, outc_ref):
    f32 = jnp.float32
    nf = nf_ref[0]
    adj = adj_ref[0]
    no = no_ref[0, 0, 0]
    ne = ne_ref[0, 0, 0]
    nmask_row = (jax.lax.broadcasted_iota(jnp.int32, (1, N), 1) < no).astype(f32)
    nmask_col = (jax.lax.broadcasted_iota(jnp.int32, (N, 1), 0) < no).astype(f32)
    emask_row = (jax.lax.broadcasted_iota(jnp.int32, (1, E), 1) < ne).astype(f32)
    emask_col = (jax.lax.broadcasted_iota(jnp.int32, (E, 1), 0) < ne).astype(f32)

    hn = jnp.dot(nf, Wn_ref[...], preferred_element_type=f32)
    adj_m = adj * nmask_row
    nn_agg = jnp.dot(adj_m, hn, preferred_element_type=f32) / (
        jnp.sum(adj_m, axis=1, keepdims=True) + 1e-6)

    src_col = eit_ref[0, :, 0:1]
    dst_col = eit_ref[0, :, 1:2]
    key = src_col * KMAX + dst_col
    g = jnp.zeros((E, DE), f32)
    for c in range(4):
        ids = jax.lax.broadcasted_iota(jnp.int32, (1, 1024), 1) + c * 1024
        oh = (key == ids).astype(f32)
        g = g + jnp.dot(oh, ef_ref[0, c * 1024:(c + 1) * 1024, :],
                        preferred_element_type=f32)
    he = jnp.dot(g, We_ref[...], preferred_element_type=f32) * emask_col

    src_row = ei_ref[0, 0:1, :]
    dst_row = ei_ref[0, 1:2, :]
    iota_n = jax.lax.broadcasted_iota(jnp.int32, (N, 1), 0)
    SD = ((iota_n == src_row).astype(f32) +
          (iota_n == dst_row).astype(f32)) * emask_row
    deg = jnp.sum(SD, axis=1, keepdims=True)
    inc = jnp.dot(SD, he, preferred_element_type=f32) / (deg + 1e-6)

    node_emb = jnp.concatenate(
        [jax.nn.relu(nn_agg), jax.nn.relu(inc)], axis=1) * nmask_col

    ladj_m = ladj_ref[0] * emask_row
    line_agg = jnp.dot(ladj_m, he, preferred_element_type=f32) / (
        jnp.sum(ladj_m, axis=1, keepdims=True) + 1e-6)

    iota_k = jax.lax.broadcasted_iota(jnp.int32, (1, KMAX), 1)
    OH_ep = ((src_col == iota_k).astype(f32) +
             (dst_col == iota_k).astype(f32))
    ep = 0.5 * jnp.dot(OH_ep, hn[:KMAX, :], preferred_element_type=f32)

    ee = jnp.concatenate([jax.nn.relu(ep), jax.nn.relu(line_agg)], axis=1)

    i0 = pairs_ref[0, :, 0:1]
    i1 = pairs_ref[0, :, 1:2]
    M = ((i0 == src_row).astype(f32) * (i1 == dst_row).astype(f32)) * emask_row
    ee_pair = jnp.dot(M, ee, preferred_element_type=f32)

    iota_nr = jax.lax.broadcasted_iota(jnp.int32, (1, N), 1)
    O = (i0 == iota_nr).astype(f32) + (i1 == iota_nr).astype(f32)
    pair_emb = jnp.dot(O, node_emb, preferred_element_type=f32)

    cls_in = jnp.concatenate([pair_emb, ee_pair], axis=1)
    for W1_ref, b1_ref, W2_ref, b2_ref, o_ref in (
            (W1a_ref, b1a_ref, W2a_ref, b2a_ref, outa_ref),
            (W1b_ref, b1b_ref, W2b_ref, b2b_ref, outb_ref),
            (W1c_ref, b1c_ref, W2c_ref, b2c_ref, outc_ref)):
        h = jax.nn.relu(jnp.dot(cls_in, W1_ref[...], preferred_element_type=f32)
                        + b1_ref[...])
        o_ref[0] = jnp.dot(h, W2_ref[...], preferred_element_type=f32) + b2_ref[...]


def kernel(concatenated_node_features, interaction_feature, adj_mat,
           line_adj_mat, nenn_edge_index, object_pairs, num_obj,
           nenn_num_edges, W_node, W_edge, lr_W1, lr_b1, lr_W2, lr_b2,
           scr_W1, scr_b1, scr_W2, scr_b2, mr_W1, mr_b1, mr_W2, mr_b2):
    f32 = jnp.float32
    ef_flat = interaction_feature[:, :KMAX, :KMAX, :].reshape(B, KMAX * KMAX, DE)
    ei_t = jnp.transpose(nenn_edge_index, (0, 2, 1))
    no2 = num_obj.reshape(B, 1, 1)
    ne2 = nenn_num_edges.reshape(B, 1, 1)

    def bspec(shape):
        return pl.BlockSpec((1,) + shape, lambda b: (b, 0, 0)[:1 + len(shape)])

    def wspec(shape):
        nd = len(shape)
        return pl.BlockSpec(shape, lambda b: (0,) * nd)

    def sspec():
        return pl.BlockSpec((1, 1, 1), lambda b: (b, 0, 0),
                            memory_space=pltpu.SMEM)

    def mlp_specs(odim):
        return [wspec((CIN, 128)), wspec((1, 128)), wspec((128, odim)),
                wspec((1, odim))]

    outs = pl.pallas_call(
        _nenn_kernel,
        grid=(B,),
        in_specs=[
            sspec(), sspec(),
            bspec((N, DN)), bspec((N, N)), bspec((E, E)), bspec((2, E)),
            bspec((E, 2)), bspec((KMAX * KMAX, DE)), bspec((P, 2)),
            wspec((DN, WN)), wspec((DE, WE)),
            *mlp_specs(ODIMS[0]), *mlp_specs(ODIMS[1]), *mlp_specs(ODIMS[2]),
        ],
        out_specs=[bspec((P, o)) for o in ODIMS],
        out_shape=[jax.ShapeDtypeStruct((B, P, o), f32) for o in ODIMS],
        compiler_params=pltpu.CompilerParams(
            dimension_semantics=("arbitrary",)),
    )(no2, ne2, concatenated_node_features, adj_mat, line_adj_mat,
      nenn_edge_index, ei_t, ef_flat, object_pairs, W_node, W_edge,
      lr_W1, lr_b1[None, :], lr_W2, lr_b2[None, :],
      scr_W1, scr_b1[None, :], scr_W2, scr_b2[None, :],
      mr_W1, mr_b1[None, :], mr_W2, mr_b2[None, :])

    return (outs[0], outs[1], outs[2])
